# baseline (speedup 1.0000x reference)
import jax
import jax.numpy as jnp
from jax import lax
from jax.experimental import pallas as pl
from jax.experimental.pallas import tpu as pltpu
from jax.experimental.pallas import tpu_sc as plsc

DIM = 2048
IMG = 784
LEVELS = 256
NUM_CLASSES = 10
BATCH = 128
NSC = 8
ITEMS = 8
SC_CORES = 2
SC_SUBCORES = 16
NW = SC_CORES * SC_SUBCORES
COLS = DIM // NW
CHUNK = 8
IB = 4
LANES = 16


def _hdc_body(x_ref, pos_ref, vt_ref, am_ref, out_ref, enc_ref):
    xb = x_ref[0]
    idx = jnp.clip(jnp.round(xb * (LEVELS - 1)), 0.0, LEVELS - 1.0).astype(jnp.int32)
    lvl = jax.lax.broadcasted_iota(jnp.int32, (LEVELS, IMG), 0)
    pos = pos_ref[...]
    vt = vt_ref[...]
    for j in range(ITEMS):
        onehot = (lvl == idx[j:j + 1, :]).astype(jnp.bfloat16)
        h = jax.lax.dot_general(
            onehot, pos, (((1,), (0,)), ((), ())),
            preferred_element_type=jnp.float32)
        enc_ref[j:j + 1, :] = jnp.sum(h * vt, axis=0, keepdims=True)
    enc = enc_ref[...]
    am = am_ref[...]
    dots = jax.lax.dot_general(
        enc, am, (((1,), (1,)), ((), ())), preferred_element_type=jnp.float32)
    ne = jnp.sqrt(jnp.sum(enc * enc, axis=1, keepdims=True)) + 1e-12
    na = jnp.sqrt(jnp.sum(am * am, axis=1)).reshape(1, NUM_CLASSES) + 1e-12
    out_ref[0] = dots / ne / na


def _tc_classify(flat, position, value_table, am, nb):
    pos_bf = position.astype(jnp.bfloat16)
    out = pl.pallas_call(
        _hdc_body,
        grid=(nb // ITEMS,),
        in_specs=[
            pl.BlockSpec((1, ITEMS, IMG), lambda i: (i, 0, 0)),
            pl.BlockSpec((IMG, DIM), lambda i: (0, 0)),
            pl.BlockSpec((LEVELS, DIM), lambda i: (0, 0)),
            pl.BlockSpec((NUM_CLASSES, DIM), lambda i: (0, 0)),
        ],
        out_specs=pl.BlockSpec((1, ITEMS, NUM_CLASSES), lambda i: (i, 0, 0)),
        out_shape=jax.ShapeDtypeStruct((nb // ITEMS, ITEMS, NUM_CLASSES), jnp.float32),
        scratch_shapes=[pltpu.VMEM((ITEMS, DIM), jnp.float32)],
    )(flat.reshape(nb // ITEMS, ITEMS, IMG), pos_bf, value_table, am)
    return out.reshape(nb, NUM_CLASSES)


def _idx_body(x_ref, idx_ref):
    idx_ref[...] = jnp.clip(
        jnp.round(x_ref[...] * (LEVELS - 1)), 0.0, LEVELS - 1.0).astype(jnp.int32)


def _enc_body(nb, idx_hbm, pos_hbm, vt_hbm, out_hbm, idx_v, pos_v, vt_v, enc_v):
    c = lax.axis_index("c")
    s = lax.axis_index("s")
    wid = s * SC_CORES + c
    col0 = wid * COLS
    pltpu.sync_copy(pos_hbm.at[:, pl.ds(col0, COLS)], pos_v)
    pltpu.sync_copy(vt_hbm.at[:, pl.ds(col0, COLS)], vt_v)

    def chunk_body(ci, carry):
        pltpu.sync_copy(idx_hbm.at[pl.ds(ci * CHUNK, CHUNK), :], idx_v)

        def item_body(b, carry2):
            nvec = COLS // LANES

            def grp_body(g, accs):
                idxvec = idx_v[b, pl.ds(g * LANES, LANES)]
                for k in range(LANES):
                    t = idxvec[k]
                    p = g * LANES + k
                    accs = tuple(
                        accs[j] + vt_v[t, pl.ds(LANES * j, LANES)] *
                        pos_v[p, pl.ds(LANES * j, LANES)]
                        for j in range(nvec))
                return accs

            accs = lax.fori_loop(
                0, IMG // LANES, grp_body,
                tuple(jnp.zeros((LANES,), jnp.float32)
                      for _ in range(nvec)))
            for j in range(nvec):
                enc_v[b, pl.ds(LANES * j, LANES)] = accs[j]
            return carry2

        lax.fori_loop(0, CHUNK, item_body, 0)
        pltpu.sync_copy(
            enc_v, out_hbm.at[pl.ds(ci * CHUNK, CHUNK), pl.ds(col0, COLS)])
        return carry

    lax.fori_loop(0, nb // CHUNK, chunk_body, 0)


def _am_body(enc_ref, am_ref, out_ref):
    enc = enc_ref[...]
    am = am_ref[...]
    dots = lax.dot_general(
        enc, am, (((1,), (1,)), ((), ())), preferred_element_type=jnp.float32)
    ne = jnp.sqrt(jnp.sum(enc * enc, axis=1, keepdims=True)) + 1e-12
    na = jnp.sqrt(jnp.sum(am * am, axis=1)).reshape(1, NUM_CLASSES) + 1e-12
    out_ref[...] = dots / ne / na


def _sc_classify(flat, position, value_table, am, nb):
    idx = pl.pallas_call(
        _idx_body,
        out_shape=jax.ShapeDtypeStruct((nb, IMG), jnp.int32),
    )(flat)
    mesh = plsc.VectorSubcoreMesh(
        core_axis_name="c", subcore_axis_name="s",
        num_cores=SC_CORES, num_subcores=SC_SUBCORES)
    enc = pl.kernel(
        lambda *a: _enc_body(nb, *a),
        out_type=jax.ShapeDtypeStruct((nb, DIM), jnp.float32),
        mesh=mesh,
        compiler_params=pltpu.CompilerParams(use_tc_tiling_on_sc=False),
        scratch_types=[
            pltpu.VMEM((CHUNK, IMG), jnp.int32),
            pltpu.VMEM((IMG, COLS), jnp.float32),
            pltpu.VMEM((LEVELS, COLS), jnp.float32),
            pltpu.VMEM((CHUNK, COLS), jnp.float32),
        ],
    )(idx, position, value_table)
    return pl.pallas_call(
        _am_body,
        out_shape=jax.ShapeDtypeStruct((nb, NUM_CLASSES), jnp.float32),
    )(enc, am)


def kernel(x, position, value_table, am):
    flat = x.reshape(BATCH, IMG)
    out_sc = _sc_classify(flat[:NSC], position, value_table, am, NSC)
    out_tc = _tc_classify(flat[NSC:], position, value_table, am, BATCH - NSC)
    return jnp.concatenate([out_sc, out_tc], axis=0)


# speedup vs baseline: 1.0538x; 1.0538x over previous
import jax
import jax.numpy as jnp
from jax import lax
from jax.experimental import pallas as pl
from jax.experimental.pallas import tpu as pltpu
from jax.experimental.pallas import tpu_sc as plsc

DIM = 2048
IMG = 784
LEVELS = 256
NUM_CLASSES = 10
BATCH = 128
NSC = 16
ITEMS = 8
SC_CORES = 2
SC_SUBCORES = 16
NW = SC_CORES * SC_SUBCORES
COLS = DIM // NW
CHUNK = 16
IB = 4
LANES = 16


def _hdc_body(x_ref, pos_ref, vt_ref, am_ref, out_ref, enc_ref):
    xb = x_ref[0]
    idx = jnp.clip(jnp.round(xb * (LEVELS - 1)), 0.0, LEVELS - 1.0).astype(jnp.int32)
    lvl = jax.lax.broadcasted_iota(jnp.int32, (LEVELS, IMG), 0)
    pos = pos_ref[...]
    vt = vt_ref[...]
    for j in range(ITEMS):
        onehot = (lvl == idx[j:j + 1, :]).astype(jnp.bfloat16)
        h = jax.lax.dot_general(
            onehot, pos, (((1,), (0,)), ((), ())),
            preferred_element_type=jnp.float32)
        enc_ref[j:j + 1, :] = jnp.sum(h * vt, axis=0, keepdims=True)
    enc = enc_ref[...]
    am = am_ref[...]
    dots = jax.lax.dot_general(
        enc, am, (((1,), (1,)), ((), ())), preferred_element_type=jnp.float32)
    ne = jnp.sqrt(jnp.sum(enc * enc, axis=1, keepdims=True)) + 1e-12
    na = jnp.sqrt(jnp.sum(am * am, axis=1)).reshape(1, NUM_CLASSES) + 1e-12
    out_ref[0] = dots / ne / na


def _tc_classify(flat, position, value_table, am, nb):
    pos_bf = position.astype(jnp.bfloat16)
    out = pl.pallas_call(
        _hdc_body,
        grid=(nb // ITEMS,),
        in_specs=[
            pl.BlockSpec((1, ITEMS, IMG), lambda i: (i, 0, 0)),
            pl.BlockSpec((IMG, DIM), lambda i: (0, 0)),
            pl.BlockSpec((LEVELS, DIM), lambda i: (0, 0)),
            pl.BlockSpec((NUM_CLASSES, DIM), lambda i: (0, 0)),
        ],
        out_specs=pl.BlockSpec((1, ITEMS, NUM_CLASSES), lambda i: (i, 0, 0)),
        out_shape=jax.ShapeDtypeStruct((nb // ITEMS, ITEMS, NUM_CLASSES), jnp.float32),
        scratch_shapes=[pltpu.VMEM((ITEMS, DIM), jnp.float32)],
    )(flat.reshape(nb // ITEMS, ITEMS, IMG), pos_bf, value_table, am)
    return out.reshape(nb, NUM_CLASSES)


def _idx_body(x_ref, idx_ref):
    idx_ref[...] = jnp.clip(
        jnp.round(x_ref[...] * (LEVELS - 1)), 0.0, LEVELS - 1.0).astype(jnp.int32)


def _enc_body(nb, idx_hbm, pos_hbm, vt_hbm, out_hbm, idx_v, pos_v, vt_v, enc_v):
    c = lax.axis_index("c")
    s = lax.axis_index("s")
    wid = s * SC_CORES + c
    col0 = wid * COLS
    pltpu.sync_copy(pos_hbm.at[:, pl.ds(col0, COLS)], pos_v)
    pltpu.sync_copy(vt_hbm.at[:, pl.ds(col0, COLS)], vt_v)

    def chunk_body(ci, carry):
        pltpu.sync_copy(idx_hbm.at[pl.ds(ci * CHUNK, CHUNK), :], idx_v)

        def item_body(b, carry2):
            nvec = COLS // LANES

            def grp_body(g, accs):
                idxvec = idx_v[b, pl.ds(g * LANES, LANES)]
                for k in range(LANES):
                    t = idxvec[k]
                    p = g * LANES + k
                    accs = tuple(
                        accs[j] + vt_v[t, pl.ds(LANES * j, LANES)] *
                        pos_v[p, pl.ds(LANES * j, LANES)]
                        for j in range(nvec))
                return accs

            accs = lax.fori_loop(
                0, IMG // LANES, grp_body,
                tuple(jnp.zeros((LANES,), jnp.float32)
                      for _ in range(nvec)))
            for j in range(nvec):
                enc_v[b, pl.ds(LANES * j, LANES)] = accs[j]
            return carry2

        lax.fori_loop(0, CHUNK, item_body, 0)
        pltpu.sync_copy(
            enc_v, out_hbm.at[pl.ds(ci * CHUNK, CHUNK), pl.ds(col0, COLS)])
        return carry

    lax.fori_loop(0, nb // CHUNK, chunk_body, 0)


def _am_body(enc_ref, am_ref, out_ref):
    enc = enc_ref[...]
    am = am_ref[...]
    dots = lax.dot_general(
        enc, am, (((1,), (1,)), ((), ())), preferred_element_type=jnp.float32)
    ne = jnp.sqrt(jnp.sum(enc * enc, axis=1, keepdims=True)) + 1e-12
    na = jnp.sqrt(jnp.sum(am * am, axis=1)).reshape(1, NUM_CLASSES) + 1e-12
    out_ref[...] = dots / ne / na


def _sc_classify(flat, position, value_table, am, nb):
    idx = pl.pallas_call(
        _idx_body,
        out_shape=jax.ShapeDtypeStruct((nb, IMG), jnp.int32),
    )(flat)
    mesh = plsc.VectorSubcoreMesh(
        core_axis_name="c", subcore_axis_name="s",
        num_cores=SC_CORES, num_subcores=SC_SUBCORES)
    enc = pl.kernel(
        lambda *a: _enc_body(nb, *a),
        out_type=jax.ShapeDtypeStruct((nb, DIM), jnp.float32),
        mesh=mesh,
        compiler_params=pltpu.CompilerParams(use_tc_tiling_on_sc=False),
        scratch_types=[
            pltpu.VMEM((CHUNK, IMG), jnp.int32),
            pltpu.VMEM((IMG, COLS), jnp.float32),
            pltpu.VMEM((LEVELS, COLS), jnp.float32),
            pltpu.VMEM((CHUNK, COLS), jnp.float32),
        ],
    )(idx, position, value_table)
    return pl.pallas_call(
        _am_body,
        out_shape=jax.ShapeDtypeStruct((nb, NUM_CLASSES), jnp.float32),
    )(enc, am)


def kernel(x, position, value_table, am):
    flat = x.reshape(BATCH, IMG)
    out_sc = _sc_classify(flat[:NSC], position, value_table, am, NSC)
    out_tc = _tc_classify(flat[NSC:], position, value_table, am, BATCH - NSC)
    return jnp.concatenate([out_sc, out_tc], axis=0)
